# baseline (speedup 1.0000x reference)
import functools

import jax
import jax.numpy as jnp
import numpy as np
from jax import lax
from jax.experimental import pallas as pl
from jax.experimental.pallas import tpu as pltpu
from jax.experimental.pallas import tpu_sc as plsc

F = 26
D = 16
B = 4096
V = 3846
NPAIR = F * (F - 1) // 2
NC, NS = 2, 16
NW = NC * NS
DH = D // 2
W = 4096
CB = 512
NU = 2 * NPAIR
UBASE = NU // NW
UEXTRA = NU - UBASE * NW

_mesh = plsc.VectorSubcoreMesh(core_axis_name="c", subcore_axis_name="s")


@functools.partial(
    pl.kernel,
    out_type=jax.ShapeDtypeStruct((NPAIR, D, B), jnp.float32),
    mesh=_mesh,
    compiler_params=pltpu.CompilerParams(
        use_tc_tiling_on_sc=True, needs_layout_passes=False
    ),
    scratch_types=[
        pltpu.VMEM((2, DH, W), jnp.float32),
        pltpu.VMEM((DH, W), jnp.float32),
        pltpu.VMEM((2, DH, CB), jnp.float32),
        pltpu.VMEM((2, CB), jnp.int32),
        pltpu.VMEM((2, CB), jnp.int32),
        pltpu.SemaphoreType.DMA((2,)),
        pltpu.SemaphoreType.DMA,
        pltpu.SemaphoreType.DMA((2,)),
        pltpu.SemaphoreType.DMA((2,)),
    ],
)
def _ffm_sc(xf_hbm, tabT_hbm, out_hbm, BAr, BB, OS, XA, XB, sba, sbb, so, sx):
    wid = lax.axis_index("s") * NC + lax.axis_index("c")
    u0 = wid * UBASE + jnp.minimum(wid, UEXTRA)
    u1 = u0 + UBASE + jnp.where(wid < UEXTRA, 1, 0)
    p0 = u0 // 2

    def tri_body(t, st):
        i0, s = st
        nb = s + (F - 1 - t)
        c = jnp.logical_and(t == i0, nb <= p0)
        return jnp.where(c, t + 1, i0), jnp.where(c, nb, s)

    i0, s0 = lax.fori_loop(0, F - 1, tri_body, (jnp.int32(0), jnp.int32(0)))
    j0 = i0 + 1 + (p0 - s0)

    def fire_a(fi, fj, fh, slot):
        cola = (fi * V // 128) * 128
        pltpu.async_copy(
            tabT_hbm.at[fj, pl.ds(fh * DH, DH), pl.ds(cola, W)],
            BAr.at[slot], sba.at[slot],
        )

    fire_a(i0, j0, u0 % 2, u0 % 2)

    def unit_body(u, carry):
        i, j = carry
        h = lax.rem(u, 2)
        p = u // 2

        colb = (j * V // 128) * 128
        cpb = pltpu.async_copy(
            tabT_hbm.at[i, pl.ds(h * DH, DH), pl.ds(colb, W)], BB, sbb
        )

        adv = j == (F - 1)
        ni = jnp.where(adv, i + 1, i)
        nj = jnp.where(adv, ni + 1, j + 1)
        fi = jnp.where(h == 1, ni, i)
        fj = jnp.where(h == 1, nj, j)

        @pl.when(u + 1 < u1)
        def _():
            fire_a(fi, fj, 1 - h, 1 - h)

        cola = (i * V // 128) * 128
        ra = i * V - cola
        rb = j * V - colb

        pltpu.async_copy(xf_hbm.at[pl.ds(i * B, CB)], XA.at[0], sx.at[0])
        pltpu.async_copy(xf_hbm.at[pl.ds(j * B, CB)], XB.at[0], sx.at[0])

        pltpu.make_async_copy(
            tabT_hbm.at[0, pl.ds(0, DH), pl.ds(0, W)], BAr.at[h], sba.at[h]
        ).wait()
        cpb.wait()

        def sc_body(sc, _):
            slot = lax.rem(sc, 2)
            nxt = lax.rem(sc + 1, 2)

            @pl.when(sc + 1 < B // CB)
            def _():
                pltpu.async_copy(
                    xf_hbm.at[pl.ds(i * B + (sc + 1) * CB, CB)], XA.at[nxt], sx.at[nxt]
                )
                pltpu.async_copy(
                    xf_hbm.at[pl.ds(j * B + (sc + 1) * CB, CB)], XB.at[nxt], sx.at[nxt]
                )

            pltpu.make_async_copy(
                xf_hbm.at[pl.ds(0, CB)], XA.at[slot], sx.at[slot]
            ).wait()
            pltpu.make_async_copy(
                xf_hbm.at[pl.ds(0, CB)], XB.at[slot], sx.at[slot]
            ).wait()

            os = slot
            g = (u - u0) * (B // CB) + sc

            @pl.when(g >= 2)
            def _():
                pltpu.make_async_copy(
                    OS.at[os], out_hbm.at[0, pl.ds(0, DH), pl.ds(0, CB)], so.at[os]
                ).wait()

            for s in range(CB // D):
                va = XA[slot, pl.ds(s * D, D)] + ra
                vb = XB[slot, pl.ds(s * D, D)] + rb
                for d in range(DH):
                    dv = jnp.full((D,), d, jnp.int32)
                    a = plsc.load_gather(BAr.at[h], [dv, va])
                    b = plsc.load_gather(BB, [dv, vb])
                    OS[os, d, pl.ds(s * D, D)] = a * b

            pltpu.async_copy(
                OS.at[os],
                out_hbm.at[p, pl.ds(h * DH, DH), pl.ds(sc * CB, CB)],
                so.at[os],
            )
            return 0

        lax.fori_loop(0, B // CB, sc_body, 0)

        return jnp.where(h == 1, ni, i), jnp.where(h == 1, nj, j)

    lax.fori_loop(u0, u1, unit_body, (i0, j0))

    for os in range(2):
        pltpu.make_async_copy(
            OS.at[os], out_hbm.at[0, pl.ds(0, DH), pl.ds(0, CB)], so.at[os]
        ).wait()


def kernel(x, tables):
    tabT = jnp.transpose(tables, (0, 2, 1))
    xf = jnp.transpose(x.astype(jnp.int32)).reshape(F * B)
    out3 = _ffm_sc(xf, tabT)
    return jnp.transpose(out3, (2, 0, 1))


# speedup vs baseline: 1.6867x; 1.6867x over previous
import functools

import jax
import jax.numpy as jnp
import numpy as np
from jax import lax
from jax.experimental import pallas as pl
from jax.experimental.pallas import tpu as pltpu
from jax.experimental.pallas import tpu_sc as plsc

F = 26
D = 16
B = 4096
V = 3846
NPAIR = F * (F - 1) // 2
NC, NS = 2, 16
NW = NC * NS
DH = D // 2
W = 4096
CB = 128
NU = 2 * NPAIR
UBASE = NU // NW
UEXTRA = NU - UBASE * NW

_mesh = plsc.VectorSubcoreMesh(core_axis_name="c", subcore_axis_name="s")


@functools.partial(
    pl.kernel,
    out_type=jax.ShapeDtypeStruct((NPAIR, D, B), jnp.float32),
    mesh=_mesh,
    compiler_params=pltpu.CompilerParams(
        use_tc_tiling_on_sc=True, needs_layout_passes=False
    ),
    scratch_types=[
        pltpu.VMEM((2, DH, W), jnp.float32),
        pltpu.VMEM((DH, W), jnp.float32),
        pltpu.VMEM((2, DH, CB), jnp.float32),
        pltpu.VMEM((2, CB), jnp.int32),
        pltpu.VMEM((2, CB), jnp.int32),
        pltpu.SemaphoreType.DMA((2,)),
        pltpu.SemaphoreType.DMA,
        pltpu.SemaphoreType.DMA((2,)),
        pltpu.SemaphoreType.DMA((2,)),
    ],
)
def _ffm_sc(xf_hbm, tabT_hbm, out_hbm, BAr, BB, OS, XA, XB, sba, sbb, so, sx):
    wid = lax.axis_index("s") * NC + lax.axis_index("c")
    u0 = wid * UBASE + jnp.minimum(wid, UEXTRA)
    u1 = u0 + UBASE + jnp.where(wid < UEXTRA, 1, 0)
    p0 = u0 // 2

    def tri_body(t, st):
        i0, s = st
        nb = s + (F - 1 - t)
        c = jnp.logical_and(t == i0, nb <= p0)
        return jnp.where(c, t + 1, i0), jnp.where(c, nb, s)

    i0, s0 = lax.fori_loop(0, F - 1, tri_body, (jnp.int32(0), jnp.int32(0)))
    j0 = i0 + 1 + (p0 - s0)

    def fire_a(fi, fj, fh, slot):
        cola = (fi * V // 128) * 128
        pltpu.async_copy(
            tabT_hbm.at[fj, pl.ds(fh * DH, DH), pl.ds(cola, W)],
            BAr.at[slot], sba.at[slot],
        )

    fire_a(i0, j0, u0 % 2, u0 % 2)

    def unit_body(u, carry):
        i, j = carry
        h = lax.rem(u, 2)
        p = u // 2

        colb = (j * V // 128) * 128
        cpb = pltpu.async_copy(
            tabT_hbm.at[i, pl.ds(h * DH, DH), pl.ds(colb, W)], BB, sbb
        )

        adv = j == (F - 1)
        ni = jnp.where(adv, i + 1, i)
        nj = jnp.where(adv, ni + 1, j + 1)
        fi = jnp.where(h == 1, ni, i)
        fj = jnp.where(h == 1, nj, j)

        @pl.when(u + 1 < u1)
        def _():
            fire_a(fi, fj, 1 - h, 1 - h)

        cola = (i * V // 128) * 128
        ra = i * V - cola
        rb = j * V - colb

        pltpu.async_copy(xf_hbm.at[pl.ds(i * B, CB)], XA.at[0], sx.at[0])
        pltpu.async_copy(xf_hbm.at[pl.ds(j * B, CB)], XB.at[0], sx.at[0])

        pltpu.make_async_copy(
            tabT_hbm.at[0, pl.ds(0, DH), pl.ds(0, W)], BAr.at[h], sba.at[h]
        ).wait()
        cpb.wait()

        def sc_body(sc, _):
            slot = lax.rem(sc, 2)
            nxt = lax.rem(sc + 1, 2)

            @pl.when(sc + 1 < B // CB)
            def _():
                pltpu.async_copy(
                    xf_hbm.at[pl.ds(i * B + (sc + 1) * CB, CB)], XA.at[nxt], sx.at[nxt]
                )
                pltpu.async_copy(
                    xf_hbm.at[pl.ds(j * B + (sc + 1) * CB, CB)], XB.at[nxt], sx.at[nxt]
                )

            pltpu.make_async_copy(
                xf_hbm.at[pl.ds(0, CB)], XA.at[slot], sx.at[slot]
            ).wait()
            pltpu.make_async_copy(
                xf_hbm.at[pl.ds(0, CB)], XB.at[slot], sx.at[slot]
            ).wait()

            os = slot
            g = (u - u0) * (B // CB) + sc

            @pl.when(g >= 2)
            def _():
                pltpu.make_async_copy(
                    OS.at[os], out_hbm.at[0, pl.ds(0, DH), pl.ds(0, CB)], so.at[os]
                ).wait()

            @plsc.parallel_loop(0, CB // D, 1, unroll=2)
            def _(s):
                va = XA[slot, pl.ds(s * D, D)] + ra
                vb = XB[slot, pl.ds(s * D, D)] + rb
                for d in range(DH):
                    dv = jnp.full((D,), d, jnp.int32)
                    a = plsc.load_gather(BAr.at[h], [dv, va])
                    b = plsc.load_gather(BB, [dv, vb])
                    OS[os, d, pl.ds(s * D, D)] = a * b

            pltpu.async_copy(
                OS.at[os],
                out_hbm.at[p, pl.ds(h * DH, DH), pl.ds(sc * CB, CB)],
                so.at[os],
            )
            return 0

        lax.fori_loop(0, B // CB, sc_body, 0)

        return jnp.where(h == 1, ni, i), jnp.where(h == 1, nj, j)

    lax.fori_loop(u0, u1, unit_body, (i0, j0))

    for os in range(2):
        pltpu.make_async_copy(
            OS.at[os], out_hbm.at[0, pl.ds(0, DH), pl.ds(0, CB)], so.at[os]
        ).wait()


def kernel(x, tables):
    tabT = jnp.transpose(tables, (0, 2, 1))
    xf = jnp.transpose(x.astype(jnp.int32)).reshape(F * B)
    out3 = _ffm_sc(xf, tabT)
    return jnp.transpose(out3, (2, 0, 1))


# speedup vs baseline: 2.2899x; 1.3577x over previous
import functools

import jax
import jax.numpy as jnp
import numpy as np
from jax import lax
from jax.experimental import pallas as pl
from jax.experimental.pallas import tpu as pltpu
from jax.experimental.pallas import tpu_sc as plsc

F = 26
D = 16
B = 4096
V = 3846
NPAIR = F * (F - 1) // 2
NC, NS = 2, 16
NW = NC * NS
DH = D // 2
W = 4096
CB = 256
NU = 2 * NPAIR
UBASE = NU // NW
UEXTRA = NU - UBASE * NW

_mesh = plsc.VectorSubcoreMesh(core_axis_name="c", subcore_axis_name="s")


@functools.partial(
    pl.kernel,
    out_type=jax.ShapeDtypeStruct((NPAIR, D, B), jnp.float32),
    mesh=_mesh,
    compiler_params=pltpu.CompilerParams(
        use_tc_tiling_on_sc=True, needs_layout_passes=False
    ),
    scratch_types=[
        pltpu.VMEM((2, DH, W), jnp.float32),
        pltpu.VMEM((DH, W), jnp.float32),
        pltpu.VMEM((2, DH, CB), jnp.float32),
        pltpu.VMEM((2, CB), jnp.int32),
        pltpu.VMEM((2, CB), jnp.int32),
        pltpu.SemaphoreType.DMA((2,)),
        pltpu.SemaphoreType.DMA,
        pltpu.SemaphoreType.DMA((2,)),
        pltpu.SemaphoreType.DMA((2,)),
    ],
)
def _ffm_sc(xf_hbm, tabT_hbm, out_hbm, BAr, BB, OS, XA, XB, sba, sbb, so, sx):
    wid = lax.axis_index("s") * NC + lax.axis_index("c")
    u0 = wid * UBASE + jnp.minimum(wid, UEXTRA)
    u1 = u0 + UBASE + jnp.where(wid < UEXTRA, 1, 0)
    p0 = u0 // 2

    def tri_body(t, st):
        i0, s = st
        nb = s + (F - 1 - t)
        c = jnp.logical_and(t == i0, nb <= p0)
        return jnp.where(c, t + 1, i0), jnp.where(c, nb, s)

    i0, s0 = lax.fori_loop(0, F - 1, tri_body, (jnp.int32(0), jnp.int32(0)))
    j0 = i0 + 1 + (p0 - s0)

    def fire_a(fi, fj, fh, slot):
        cola = (fi * V // 128) * 128
        pltpu.async_copy(
            tabT_hbm.at[fj, pl.ds(fh * DH, DH), pl.ds(cola, W)],
            BAr.at[slot], sba.at[slot],
        )

    fire_a(i0, j0, u0 % 2, u0 % 2)

    def unit_body(u, carry):
        i, j = carry
        h = lax.rem(u, 2)
        p = u // 2

        colb = (j * V // 128) * 128
        cpb = pltpu.async_copy(
            tabT_hbm.at[i, pl.ds(h * DH, DH), pl.ds(colb, W)], BB, sbb
        )

        adv = j == (F - 1)
        ni = jnp.where(adv, i + 1, i)
        nj = jnp.where(adv, ni + 1, j + 1)
        fi = jnp.where(h == 1, ni, i)
        fj = jnp.where(h == 1, nj, j)

        @pl.when(u + 1 < u1)
        def _():
            fire_a(fi, fj, 1 - h, 1 - h)

        cola = (i * V // 128) * 128
        ra = i * V - cola
        rb = j * V - colb

        pltpu.async_copy(xf_hbm.at[pl.ds(i * B, CB)], XA.at[0], sx.at[0])
        pltpu.async_copy(xf_hbm.at[pl.ds(j * B, CB)], XB.at[0], sx.at[0])

        pltpu.make_async_copy(
            tabT_hbm.at[0, pl.ds(0, DH), pl.ds(0, W)], BAr.at[h], sba.at[h]
        ).wait()
        cpb.wait()

        def sc_body(sc, _):
            slot = lax.rem(sc, 2)
            nxt = lax.rem(sc + 1, 2)

            @pl.when(sc + 1 < B // CB)
            def _():
                pltpu.async_copy(
                    xf_hbm.at[pl.ds(i * B + (sc + 1) * CB, CB)], XA.at[nxt], sx.at[nxt]
                )
                pltpu.async_copy(
                    xf_hbm.at[pl.ds(j * B + (sc + 1) * CB, CB)], XB.at[nxt], sx.at[nxt]
                )

            pltpu.make_async_copy(
                xf_hbm.at[pl.ds(0, CB)], XA.at[slot], sx.at[slot]
            ).wait()
            pltpu.make_async_copy(
                xf_hbm.at[pl.ds(0, CB)], XB.at[slot], sx.at[slot]
            ).wait()

            os = slot
            g = (u - u0) * (B // CB) + sc

            @pl.when(g >= 2)
            def _():
                pltpu.make_async_copy(
                    OS.at[os], out_hbm.at[0, pl.ds(0, DH), pl.ds(0, CB)], so.at[os]
                ).wait()

            @plsc.parallel_loop(0, CB // D, 1, unroll=2)
            def _(s):
                va = XA[slot, pl.ds(s * D, D)] + ra
                vb = XB[slot, pl.ds(s * D, D)] + rb
                for d in range(DH):
                    dv = jnp.full((D,), d, jnp.int32)
                    a = plsc.load_gather(BAr.at[h], [dv, va])
                    b = plsc.load_gather(BB, [dv, vb])
                    OS[os, d, pl.ds(s * D, D)] = a * b

            pltpu.async_copy(
                OS.at[os],
                out_hbm.at[p, pl.ds(h * DH, DH), pl.ds(sc * CB, CB)],
                so.at[os],
            )
            return 0

        lax.fori_loop(0, B // CB, sc_body, 0)

        return jnp.where(h == 1, ni, i), jnp.where(h == 1, nj, j)

    lax.fori_loop(u0, u1, unit_body, (i0, j0))

    for os in range(2):
        pltpu.make_async_copy(
            OS.at[os], out_hbm.at[0, pl.ds(0, DH), pl.ds(0, CB)], so.at[os]
        ).wait()


def kernel(x, tables):
    tabT = jnp.transpose(tables, (0, 2, 1))
    xf = jnp.transpose(x.astype(jnp.int32)).reshape(F * B)
    out3 = _ffm_sc(xf, tabT)
    return jnp.transpose(out3, (2, 0, 1))


# speedup vs baseline: 2.8485x; 1.2439x over previous
import functools

import jax
import jax.numpy as jnp
import numpy as np
from jax import lax
from jax.experimental import pallas as pl
from jax.experimental.pallas import tpu as pltpu
from jax.experimental.pallas import tpu_sc as plsc

F = 26
D = 16
B = 4096
V = 3846
NPAIR = F * (F - 1) // 2
NC, NS = 2, 16
NW = NC * NS
DH = D // 2
W = 4096
CB = 512
NU = 2 * NPAIR
UBASE = NU // NW
UEXTRA = NU - UBASE * NW

_mesh = plsc.VectorSubcoreMesh(core_axis_name="c", subcore_axis_name="s")


@functools.partial(
    pl.kernel,
    out_type=jax.ShapeDtypeStruct((NPAIR, D, B), jnp.float32),
    mesh=_mesh,
    compiler_params=pltpu.CompilerParams(
        use_tc_tiling_on_sc=True, needs_layout_passes=False
    ),
    scratch_types=[
        pltpu.VMEM((2, DH, W), jnp.float32),
        pltpu.VMEM((DH, W), jnp.float32),
        pltpu.VMEM((2, DH, CB), jnp.float32),
        pltpu.VMEM((2, CB), jnp.int32),
        pltpu.VMEM((2, CB), jnp.int32),
        pltpu.SemaphoreType.DMA((2,)),
        pltpu.SemaphoreType.DMA,
        pltpu.SemaphoreType.DMA((2,)),
        pltpu.SemaphoreType.DMA((2,)),
    ],
)
def _ffm_sc(xf_hbm, tabT_hbm, out_hbm, BAr, BB, OS, XA, XB, sba, sbb, so, sx):
    wid = lax.axis_index("s") * NC + lax.axis_index("c")
    u0 = wid * UBASE + jnp.minimum(wid, UEXTRA)
    u1 = u0 + UBASE + jnp.where(wid < UEXTRA, 1, 0)
    p0 = u0 // 2

    def tri_body(t, st):
        i0, s = st
        nb = s + (F - 1 - t)
        c = jnp.logical_and(t == i0, nb <= p0)
        return jnp.where(c, t + 1, i0), jnp.where(c, nb, s)

    i0, s0 = lax.fori_loop(0, F - 1, tri_body, (jnp.int32(0), jnp.int32(0)))
    j0 = i0 + 1 + (p0 - s0)

    def fire_a(fi, fj, fh, slot):
        cola = (fi * V // 128) * 128
        pltpu.async_copy(
            tabT_hbm.at[fj, pl.ds(fh * DH, DH), pl.ds(cola, W)],
            BAr.at[slot], sba.at[slot],
        )

    fire_a(i0, j0, u0 % 2, u0 % 2)

    def unit_body(u, carry):
        i, j = carry
        h = lax.rem(u, 2)
        p = u // 2

        colb = (j * V // 128) * 128
        cpb = pltpu.async_copy(
            tabT_hbm.at[i, pl.ds(h * DH, DH), pl.ds(colb, W)], BB, sbb
        )

        adv = j == (F - 1)
        ni = jnp.where(adv, i + 1, i)
        nj = jnp.where(adv, ni + 1, j + 1)
        fi = jnp.where(h == 1, ni, i)
        fj = jnp.where(h == 1, nj, j)

        @pl.when(u + 1 < u1)
        def _():
            fire_a(fi, fj, 1 - h, 1 - h)

        cola = (i * V // 128) * 128
        ra = i * V - cola
        rb = j * V - colb

        pltpu.async_copy(xf_hbm.at[pl.ds(i * B, CB)], XA.at[0], sx.at[0])
        pltpu.async_copy(xf_hbm.at[pl.ds(j * B, CB)], XB.at[0], sx.at[0])

        pltpu.make_async_copy(
            tabT_hbm.at[0, pl.ds(0, DH), pl.ds(0, W)], BAr.at[h], sba.at[h]
        ).wait()
        cpb.wait()

        def sc_body(sc, _):
            slot = lax.rem(sc, 2)
            nxt = lax.rem(sc + 1, 2)

            @pl.when(sc + 1 < B // CB)
            def _():
                pltpu.async_copy(
                    xf_hbm.at[pl.ds(i * B + (sc + 1) * CB, CB)], XA.at[nxt], sx.at[nxt]
                )
                pltpu.async_copy(
                    xf_hbm.at[pl.ds(j * B + (sc + 1) * CB, CB)], XB.at[nxt], sx.at[nxt]
                )

            pltpu.make_async_copy(
                xf_hbm.at[pl.ds(0, CB)], XA.at[slot], sx.at[slot]
            ).wait()
            pltpu.make_async_copy(
                xf_hbm.at[pl.ds(0, CB)], XB.at[slot], sx.at[slot]
            ).wait()

            os = slot
            g = (u - u0) * (B // CB) + sc

            @pl.when(g >= 2)
            def _():
                pltpu.make_async_copy(
                    OS.at[os], out_hbm.at[0, pl.ds(0, DH), pl.ds(0, CB)], so.at[os]
                ).wait()

            @plsc.parallel_loop(0, CB // D, 1, unroll=2)
            def _(s):
                va = XA[slot, pl.ds(s * D, D)] + ra
                vb = XB[slot, pl.ds(s * D, D)] + rb
                for d in range(DH):
                    dv = jnp.full((D,), d, jnp.int32)
                    a = plsc.load_gather(BAr.at[h], [dv, va])
                    b = plsc.load_gather(BB, [dv, vb])
                    OS[os, d, pl.ds(s * D, D)] = a * b

            pltpu.async_copy(
                OS.at[os],
                out_hbm.at[p, pl.ds(h * DH, DH), pl.ds(sc * CB, CB)],
                so.at[os],
            )
            return 0

        lax.fori_loop(0, B // CB, sc_body, 0)

        return jnp.where(h == 1, ni, i), jnp.where(h == 1, nj, j)

    lax.fori_loop(u0, u1, unit_body, (i0, j0))

    for os in range(2):
        pltpu.make_async_copy(
            OS.at[os], out_hbm.at[0, pl.ds(0, DH), pl.ds(0, CB)], so.at[os]
        ).wait()


def kernel(x, tables):
    tabT = jnp.transpose(tables, (0, 2, 1))
    xf = jnp.transpose(x.astype(jnp.int32)).reshape(F * B)
    out3 = _ffm_sc(xf, tabT)
    return jnp.transpose(out3, (2, 0, 1))
